# Initial kernel scaffold; baseline (speedup 1.0000x reference)
import jax
import jax.numpy as jnp
from jax.experimental import pallas as pl


def kernel(x, edge_index, edge_attr, recipe, batch, W_in, b_in, W_ee, b_ee, gat_W, att_src, att_dst, gat_We, att_edge, gat_b, W_r1, b_r1, W_r2, b_r2):
    raise NotImplementedError("write your pallas kernel here")


# speedup vs baseline: 7.0440x; 7.0440x over previous
import functools

import jax
import jax.numpy as jnp
from jax import lax
from jax.experimental import pallas as pl
from jax.experimental.pallas import tpu as pltpu
from jax.experimental.pallas import tpu_sc as plsc

N = 10000
E = 160000
DN = 256
DE = 16
DR = 64
HID = 512
NH = 8
HC = 64
L = 3
G = 8

NC = 2
NS = 16
NW = NC * NS

E2 = 163840
EPW = E2 // NW
N2 = 10112
NPS = N2 // NS

BN = 400
BE = 2048
CE = 128
CK2 = 128
CB = 64
NBN = N // BN
NEG = -1e30

_mesh = None


def _get_mesh():
    global _mesh
    if _mesh is None:
        _mesh = plsc.VectorSubcoreMesh(
            core_axis_name="c", subcore_axis_name="s",
            num_cores=NC, num_subcores=NS)
    return _mesh


def _k1_body(batch_ref, x_ref, recipe_ref, Wx_ref, Wr_ref, b_ref, h_ref):
    bvals = batch_ref[0, 0, :]
    onehot = (bvals[:, None] == lax.broadcasted_iota(jnp.int32, (BN, G), 1))
    onehot = onehot.astype(jnp.float32)
    Rw = jnp.dot(recipe_ref[...], Wr_ref[...],
                 preferred_element_type=jnp.float32) + b_ref[...]
    acc = jnp.dot(x_ref[...], Wx_ref[...], preferred_element_type=jnp.float32)
    acc = acc + jnp.dot(onehot, Rw, preferred_element_type=jnp.float32)
    h_ref[...] = jnp.maximum(acc, 0.0)


def _k1(x, batch3d, recipe, Wx, Wr, b_in2d):
    return pl.pallas_call(
        _k1_body,
        grid=(NBN,),
        in_specs=[
            pl.BlockSpec((1, 1, BN), lambda i: (i, 0, 0)),
            pl.BlockSpec((BN, DN), lambda i: (i, 0)),
            pl.BlockSpec((G, DR), lambda i: (0, 0)),
            pl.BlockSpec((DN, HID), lambda i: (0, 0)),
            pl.BlockSpec((DR, HID), lambda i: (0, 0)),
            pl.BlockSpec((1, HID), lambda i: (0, 0)),
        ],
        out_specs=pl.BlockSpec((BN, HID), lambda i: (i, 0)),
        out_shape=jax.ShapeDtypeStruct((N, HID), jnp.float32),
    )(batch3d, x, recipe, Wx, Wr, b_in2d)


def _k1b_body(ea_ref, U_ref, c_ref, pay_ref, ae32_ref, me_ref):
    i = pl.program_id(0)
    ae = jnp.dot(ea_ref[...], U_ref[...],
                 preferred_element_type=jnp.float32) + c_ref[...]
    ones = jnp.ones((BE, 1), jnp.float32)
    pay_ref[...] = jnp.concatenate(
        [ae, ones, jnp.zeros((BE, 103), jnp.float32)], axis=1)
    ae32_ref[...] = jnp.concatenate(
        [ae, ones, jnp.zeros((BE, 7), jnp.float32)], axis=1)
    rows = []
    for l in range(L):
        m = jnp.max(ae[:, 8 * l:8 * l + 8], axis=0, keepdims=True)
        rows.append(jnp.concatenate(
            [m, jnp.full((1, 8), NEG, jnp.float32)], axis=1))
    blk = jnp.concatenate(rows, axis=0)

    @pl.when(i == 0)
    def _():
        me_ref[...] = jnp.full((L, 16), NEG, jnp.float32)

    me_ref[...] = jnp.maximum(me_ref[...], blk)


def _k1b(ea_p, U24, c24row):
    return pl.pallas_call(
        _k1b_body,
        grid=(E2 // BE,),
        in_specs=[
            pl.BlockSpec((BE, DE), lambda i: (i, 0)),
            pl.BlockSpec((DE, L * NH), lambda i: (0, 0)),
            pl.BlockSpec((1, L * NH), lambda i: (0, 0)),
        ],
        out_specs=[
            pl.BlockSpec((BE, 128), lambda i: (i, 0)),
            pl.BlockSpec((BE, 32), lambda i: (i, 0)),
            pl.BlockSpec((L, 16), lambda i: (0, 0)),
        ],
        out_shape=[
            jax.ShapeDtypeStruct((E2, 128), jnp.float32),
            jax.ShapeDtypeStruct((E2, 32), jnp.float32),
            jax.ShapeDtypeStruct((L, 16), jnp.float32),
        ],
    )(ea_p, U24, c24row)


def _k3_body(l, h_ref, W_ref, Asd_ref, me_ref, acc32a_ref, acc32b_ref,
             xh_ref, asd_ref, msd_ref, c16_ref):
    i = pl.program_id(0)
    j = pl.program_id(1)
    asd = jnp.dot(h_ref[...], Asd_ref[...], preferred_element_type=jnp.float32)
    asd_ref[...] = asd

    @pl.when(j < 4)
    def _():
        xh_ref[0] = jnp.dot(h_ref[...], W_ref[...],
                            preferred_element_type=jnp.float32)

    @pl.when(j == 4)
    def _():
        xh_ref[0] = jnp.concatenate(
            [asd, jnp.zeros((BN, 112), jnp.float32)], axis=1)

    a32 = acc32a_ref[0] + acc32b_ref[0]
    deg = jnp.maximum(a32[:, 24:25], 1.0)
    lae = a32[:, 8 * l:8 * l + 8] / deg
    mloop = jnp.max(lae, axis=0, keepdims=True)
    blk = jnp.max(asd, axis=0, keepdims=True)
    ml16 = jnp.concatenate([mloop, jnp.full((1, 8), NEG, jnp.float32)], axis=1)

    @pl.when((i == 0) & (j == 0))
    def _():
        msd_ref[...] = jnp.full((1, 32), NEG, jnp.float32)

    cur = msd_ref[...]
    cur = jnp.maximum(cur, jnp.concatenate([blk, ml16], axis=1))
    msd_ref[...] = cur
    ms = cur[:, 0:8]
    md = cur[:, 8:16]
    ml = cur[:, 16:24]
    me = me_ref[:, 0:8]
    c8 = ms + md + jnp.maximum(me, ml)
    c8 = jnp.where(c8 > 0, c8, 0.2 * c8)
    c16_ref[...] = jnp.concatenate(
        [c8, jnp.zeros((1, 8), jnp.float32)], axis=1)


def _k3(l, h, gW_l, Asd_l, me_l, acc32P):
    return pl.pallas_call(
        functools.partial(_k3_body, l),
        grid=(NBN, 5),
        in_specs=[
            pl.BlockSpec((BN, HID), lambda i, j: (i, 0)),
            pl.BlockSpec((HID, 128), lambda i, j: (0, lax.min(j, 3))),
            pl.BlockSpec((HID, 16), lambda i, j: (0, 0)),
            pl.BlockSpec((1, 16), lambda i, j: (0, 0)),
            pl.BlockSpec((1, BN, 128), lambda i, j: (0, i, 0)),
            pl.BlockSpec((1, BN, 128), lambda i, j: (1, i, 0)),
        ],
        out_specs=[
            pl.BlockSpec((1, BN, 128), lambda i, j: (j, i, 0)),
            pl.BlockSpec((BN, 16), lambda i, j: (i, 0)),
            pl.BlockSpec((1, 32), lambda i, j: (0, 0)),
            pl.BlockSpec((1, 16), lambda i, j: (0, 0)),
        ],
        out_shape=[
            jax.ShapeDtypeStruct((5, N, 128), jnp.float32),
            jax.ShapeDtypeStruct((N, 16), jnp.float32),
            jax.ShapeDtypeStruct((1, 32), jnp.float32),
            jax.ShapeDtypeStruct((1, 16), jnp.float32),
        ],
    )(h, gW_l, Asd_l, me_l, acc32P, acc32P)


def _k6_body(l, accPa_ref, accPb_ref, xh_ref, asd_ref,
             acc32a_ref, acc32b_ref, c16_ref, b_ref, h_ref):
    acc = [accPa_ref[0, c] + accPb_ref[0, c] for c in range(4)]
    xh = [xh_ref[c] for c in range(4)]
    dns = accPa_ref[0, 4] + accPb_ref[0, 4]

    a32 = acc32a_ref[0] + acc32b_ref[0]
    deg = jnp.maximum(a32[:, 24:25], 1.0)
    lae = a32[:, 8 * l:8 * l + 8] / deg
    asd = asd_ref[...]
    lself = asd[:, 0:8] + asd[:, 8:16] + lae
    lself = jnp.where(lself > 0, lself, 0.2 * lself)
    p_self = jnp.exp(lself - c16_ref[:, 0:8])
    denom = dns[:, 0:8] + p_self + 1e-16

    cols = []
    for h8 in range(NH):
        c = h8 // 2
        off = 64 * (h8 % 2)
        num = acc[c][:, off:off + 64] + xh[c][:, off:off + 64] * p_self[:, h8:h8 + 1]
        cols.append(num / denom[:, h8:h8 + 1])
    out = jnp.concatenate(cols, axis=1) + b_ref[...]
    h_ref[...] = jnp.maximum(out, 0.0)


def _k6(l, accP, xh5, asd, acc32P, c16, b_l2d):
    return pl.pallas_call(
        functools.partial(_k6_body, l),
        grid=(NBN,),
        in_specs=[
            pl.BlockSpec((1, 5, BN, 128), lambda i: (0, 0, i, 0)),
            pl.BlockSpec((1, 5, BN, 128), lambda i: (1, 0, i, 0)),
            pl.BlockSpec((4, BN, 128), lambda i: (0, i, 0)),
            pl.BlockSpec((BN, 16), lambda i: (i, 0)),
            pl.BlockSpec((1, BN, 128), lambda i: (0, i, 0)),
            pl.BlockSpec((1, BN, 128), lambda i: (1, i, 0)),
            pl.BlockSpec((1, 16), lambda i: (0, 0)),
            pl.BlockSpec((1, HID), lambda i: (0, 0)),
        ],
        out_specs=pl.BlockSpec((BN, HID), lambda i: (i, 0)),
        out_shape=jax.ShapeDtypeStruct((N, HID), jnp.float32),
    )(accP, accP, xh5, asd, acc32P, acc32P, c16, b_l2d)


def _k7_body(batch_ref, h_ref, W1_ref, b1_ref, W2_ref, b2_ref,
             g_ref, out_ref):
    i = pl.program_id(0)
    bvals = batch_ref[0, 0, :]
    onehot = (bvals[:, None] == lax.broadcasted_iota(jnp.int32, (BN, G), 1))
    onehot = onehot.astype(jnp.float32)
    gpart = lax.dot_general(onehot, h_ref[...], (((0,), (0,)), ((), ())),
                            preferred_element_type=jnp.float32)

    @pl.when(i == 0)
    def _():
        g_ref[...] = jnp.zeros((G, HID), jnp.float32)

    g = g_ref[...] + gpart
    g_ref[...] = g
    r = jnp.maximum(jnp.dot(g, W1_ref[...],
                            preferred_element_type=jnp.float32) + b1_ref[...],
                    0.0)
    out_ref[...] = jnp.dot(r, W2_ref[...],
                           preferred_element_type=jnp.float32) + b2_ref[...]


def _k7(h, batch3d, W_r1, b_r1_2d, W_r2p, b_r2p):
    return pl.pallas_call(
        _k7_body,
        grid=(NBN,),
        in_specs=[
            pl.BlockSpec((1, 1, BN), lambda i: (i, 0, 0)),
            pl.BlockSpec((BN, HID), lambda i: (i, 0)),
            pl.BlockSpec((HID, HID), lambda i: (0, 0)),
            pl.BlockSpec((1, HID), lambda i: (0, 0)),
            pl.BlockSpec((HID, 128), lambda i: (0, 0)),
            pl.BlockSpec((1, 128), lambda i: (0, 0)),
        ],
        out_specs=[
            pl.BlockSpec((G, HID), lambda i: (0, 0)),
            pl.BlockSpec((G, 128), lambda i: (0, 0)),
        ],
        out_shape=[
            jax.ShapeDtypeStruct((G, HID), jnp.float32),
            jax.ShapeDtypeStruct((G, 128), jnp.float32),
        ],
    )(batch3d, h, W_r1, b_r1_2d, W_r2p, b_r2p)


def _k2_body(pay_hbm, dst_hbm, zeros_hbm, out_hbm, pay, didx, sacc):
    cid = lax.axis_index("c")
    sid = lax.axis_index("s")
    wid = sid * NC + cid
    pltpu.sync_copy(zeros_hbm.at[pl.ds(sid * NPS, NPS)],
                    sacc.at[pl.ds(sid * NPS, NPS)])
    plsc.subcore_barrier()

    def chunk(k, _):
        base = wid * EPW + k * CK2
        pltpu.sync_copy(pay_hbm.at[pl.ds(base, CK2)], pay)
        pltpu.sync_copy(dst_hbm.at[pl.ds(base, CK2)], didx.at[0])
        pltpu.sync_copy(pay, sacc.at[didx.at[0]], add=True)
        return _

    lax.fori_loop(0, EPW // CK2, chunk, 0)
    plsc.subcore_barrier()
    pltpu.sync_copy(sacc.at[pl.ds(sid * NPS, NPS)],
                    out_hbm.at[cid, pl.ds(sid * NPS, NPS)])


def _k2(payload, dst_p, zeros128):
    f = pl.kernel(
        _k2_body,
        out_type=jax.ShapeDtypeStruct((NC, N2, 128), jnp.float32),
        mesh=_get_mesh(),
        compiler_params=pltpu.CompilerParams(needs_layout_passes=False),
        scratch_types=[
            pltpu.VMEM((CK2, 128), jnp.float32),
            pltpu.VMEM((1, CK2), jnp.int32),
            pltpu.VMEM_SHARED((N2, 128), jnp.float32),
        ],
    )
    return f(payload, dst_p, zeros128)


def _k4_body(l, xh_hbm, pay_hbm, src_hbm, dst_hbm, c16_hbm,
             p_hbm, sb, abuf, pay, pnar, sidx, didx, gs, gd,
             cvec, sem):
    cid = lax.axis_index("c")
    sid = lax.axis_index("s")
    wid = sid * NC + cid
    pltpu.sync_copy(c16_hbm.at[0], cvec)

    iota = lax.iota(jnp.int32, 16)
    chs = [plsc.load_gather(cvec, [jnp.full((16,), h, jnp.int32)])
           for h in range(NH)]

    def chunk(k, _):
        base = wid * EPW + k * CE
        pltpu.sync_copy(src_hbm.at[pl.ds(base, CE)], sidx)
        pltpu.sync_copy(dst_hbm.at[pl.ds(base, CE)], didx)
        pltpu.sync_copy(pay_hbm.at[pl.ds(base, CE)], pay)

        def shift(g, _2):
            gs[pl.ds(g * 16, 16)] = sidx[pl.ds(g * 16, 16)] + 4 * N
            gd[pl.ds(g * 16, 16)] = didx[pl.ds(g * 16, 16)] + 4 * N
            return _2

        lax.fori_loop(0, CE // 16, shift, 0)

        pltpu.async_copy(xh_hbm.at[gs], sb, sem).wait()
        for h in range(NH):
            hh = jnp.full((16,), h, jnp.int32)

            def ext_s(g, _2):
                row = g * 16 + iota
                plsc.store_scatter(abuf, [row, hh],
                                   plsc.load_gather(sb, [row, hh]))
                return _2

            lax.fori_loop(0, CE // 16, ext_s, 0)

        pltpu.async_copy(xh_hbm.at[gd], sb, sem).wait()
        for h in range(NH):
            hd = jnp.full((16,), 8 + h, jnp.int32)

            def ext_d(g, _2):
                row = g * 16 + iota
                plsc.store_scatter(abuf, [row, hd],
                                   plsc.load_gather(sb, [row, hd]))
                return _2

            lax.fori_loop(0, CE // 16, ext_d, 0)

        for h in range(NH):
            hsrc = jnp.full((16,), h, jnp.int32)
            hdst = jnp.full((16,), 8 + h, jnp.int32)
            hae = jnp.full((16,), 8 * l + h, jnp.int32)
            ch = chs[h]

            def grp(g, _2):
                row = g * 16 + iota
                a_s = plsc.load_gather(abuf, [row, hsrc])
                a_d = plsc.load_gather(abuf, [row, hdst])
                ae = plsc.load_gather(pay, [row, hae])
                t = a_s + a_d + ae
                t = jnp.where(t > 0, t, 0.2 * t)
                pv = jnp.exp(t - ch)
                plsc.store_scatter(pnar, [row, hsrc], pv)
                return _2

            lax.fori_loop(0, CE // 16, grp, 0)

        pltpu.sync_copy(pnar, p_hbm.at[pl.ds(base, CE)])
        return _

    lax.fori_loop(0, EPW // CE, chunk, 0)


def _k4(l, xh5flat, ae32, src_p, dst_p, c16):
    f = pl.kernel(
        functools.partial(_k4_body, l),
        out_type=jax.ShapeDtypeStruct((E2, 16), jnp.float32),
        mesh=_get_mesh(),
        compiler_params=pltpu.CompilerParams(needs_layout_passes=False),
        scratch_types=[
            pltpu.VMEM((CE, 128), jnp.float32),
            pltpu.VMEM((CE, 16), jnp.float32),
            pltpu.VMEM((CE, 32), jnp.float32),
            pltpu.VMEM((CE, 16), jnp.float32),
            pltpu.VMEM((CE,), jnp.int32),
            pltpu.VMEM((CE,), jnp.int32),
            pltpu.VMEM((CE,), jnp.int32),
            pltpu.VMEM((CE,), jnp.int32),
            pltpu.VMEM((16,), jnp.float32),
            pltpu.SemaphoreType.DMA,
        ],
    )
    return f(xh5flat, ae32, src_p, dst_p, c16)


def _k5_body(xh_hbm, p_hbm, src_hbm, dst_hbm, zeros_hbm, out_hbm,
             rows, pch, sidx, didx, gidx, sacc, sem):
    cid = lax.axis_index("c")
    sid = lax.axis_index("s")
    wid = sid * NC + cid

    for cc in range(5):
        pltpu.sync_copy(zeros_hbm.at[pl.ds(sid * NPS, NPS)],
                        sacc.at[pl.ds(sid * NPS, NPS)])
        plsc.subcore_barrier()

        if cc < 4:
            def chunk(k, _):
                base = wid * EPW + k * CB
                pltpu.sync_copy(src_hbm.at[pl.ds(base, CB)], sidx)
                pltpu.sync_copy(dst_hbm.at[pl.ds(base, CB)], didx)
                pltpu.sync_copy(p_hbm.at[pl.ds(base, CB)], pch)

                def shift(g, _2):
                    gidx[pl.ds(g * 16, 16)] = sidx[pl.ds(g * 16, 16)] + cc * N
                    return _2

                lax.fori_loop(0, CB // 16, shift, 0)
                pltpu.async_copy(xh_hbm.at[gidx], rows, sem).wait()

                def edge(e, _2):
                    p0 = plsc.load_gather(
                        pch, [jnp.full((16,), e, jnp.int32),
                              jnp.full((16,), 2 * cc, jnp.int32)])
                    p1 = plsc.load_gather(
                        pch, [jnp.full((16,), e, jnp.int32),
                              jnp.full((16,), 2 * cc + 1, jnp.int32)])
                    for j in range(8):
                        pj = p0 if j < 4 else p1
                        rows[e, pl.ds(j * 16, 16)] = (
                            rows[e, pl.ds(j * 16, 16)] * pj)
                    return _2

                lax.fori_loop(0, CB, edge, 0)
                pltpu.sync_copy(rows, sacc.at[didx], add=True)
                return _

            lax.fori_loop(0, EPW // CB, chunk, 0)
        else:
            def chunk4(k, _):
                base = wid * EPW + k * CB
                pltpu.sync_copy(dst_hbm.at[pl.ds(base, CB)], didx)
                pltpu.sync_copy(p_hbm.at[pl.ds(base, CB)], pch)

                def edge4(e, _2):
                    rows[e, pl.ds(0, 16)] = pch[e, :]
                    return _2

                lax.fori_loop(0, CB, edge4, 0)
                pltpu.sync_copy(rows, sacc.at[didx], add=True)
                return _

            lax.fori_loop(0, EPW // CB, chunk4, 0)

        plsc.subcore_barrier()
        pltpu.sync_copy(sacc.at[pl.ds(sid * NPS, NPS)],
                        out_hbm.at[cid, cc, pl.ds(sid * NPS, NPS)])
        plsc.subcore_barrier()


def _k5(xh4flat, p, src_p, dst_p, zeros128):
    f = pl.kernel(
        _k5_body,
        out_type=jax.ShapeDtypeStruct((NC, 5, N2, 128), jnp.float32),
        mesh=_get_mesh(),
        compiler_params=pltpu.CompilerParams(needs_layout_passes=False),
        scratch_types=[
            pltpu.VMEM((CB, 128), jnp.float32),
            pltpu.VMEM((CB, 16), jnp.float32),
            pltpu.VMEM((CB,), jnp.int32),
            pltpu.VMEM((CB,), jnp.int32),
            pltpu.VMEM((CB,), jnp.int32),
            pltpu.VMEM_SHARED((N2, 128), jnp.float32),
            pltpu.SemaphoreType.DMA,
        ],
    )
    return f(xh4flat, p, src_p, dst_p, zeros128)


def kernel(x, edge_index, edge_attr, recipe, batch, W_in, b_in, W_ee, b_ee,
           gat_W, att_src, att_dst, gat_We, att_edge, gat_b, W_r1, b_r1,
           W_r2, b_r2):
    f32 = jnp.float32
    gw = gat_W.reshape(L, HID, NH, HC)
    A_s = jnp.einsum('lkhc,lhc->lkh', gw, att_src)
    A_d = jnp.einsum('lkhc,lhc->lkh', gw, att_dst)
    A_sd = jnp.concatenate([A_s, A_d], axis=-1)
    V = jnp.einsum('lkhc,lhc->lkh', gat_We.reshape(L, HID, NH, HC), att_edge)
    U = jnp.einsum('dk,lkh->ldh', W_ee, V)
    U24 = jnp.moveaxis(U, 0, 1).reshape(DE, L * NH)
    c24row = jnp.einsum('k,lkh->lh', b_ee, V).reshape(1, L * NH)

    Wx = W_in[:DN]
    Wr = W_in[DN:]
    b_in2d = b_in.reshape(1, HID)
    batch3d = batch.astype(jnp.int32).reshape(NBN, 1, BN)
    W_r2p = jnp.pad(W_r2, ((0, 0), (0, 127)))
    b_r2p = jnp.pad(b_r2.reshape(1, 1), ((0, 0), (0, 127)))

    src_p = jnp.concatenate(
        [edge_index[0].astype(jnp.int32), jnp.zeros((E2 - E,), jnp.int32)])
    dst_p = jnp.concatenate(
        [edge_index[1].astype(jnp.int32), jnp.full((E2 - E,), N, jnp.int32)])
    ea_p = jnp.concatenate(
        [edge_attr, jnp.zeros((E2 - E, DE), f32)], axis=0)

    zeros128 = jnp.zeros((N2, 128), f32)

    h = _k1(x, batch3d, recipe, Wx, Wr, b_in2d)
    payload, ae32, m_e = _k1b(ea_p, U24, c24row)
    acc32P = _k2(payload, dst_p, zeros128)

    for l in range(L):
        xh5, asd, _msd, c16 = _k3(l, h, gat_W[l], A_sd[l],
                                  m_e[l:l + 1], acc32P)
        xh5flat = xh5.reshape(5 * N, 128)
        p = _k4(l, xh5flat, ae32, src_p, dst_p, c16)
        accP = _k5(xh5flat, p, src_p, dst_p, zeros128)
        h = _k6(l, accP, xh5, asd, acc32P, c16,
                gat_b[l].reshape(1, HID))

    _g, out = _k7(h, batch3d, W_r1, b_r1.reshape(1, HID), W_r2p, b_r2p)
    return out[:, :1]


# speedup vs baseline: 7.9248x; 1.1250x over previous
import functools

import jax
import jax.numpy as jnp
from jax import lax
from jax.experimental import pallas as pl
from jax.experimental.pallas import tpu as pltpu
from jax.experimental.pallas import tpu_sc as plsc

N = 10000
E = 160000
DN = 256
DE = 16
DR = 64
HID = 512
NH = 8
HC = 64
L = 3
G = 8

NC = 2
NS = 16
NW = NC * NS

E2 = 163840
EPW = E2 // NW
N2 = 10112
NPS = N2 // NS

BN = 400
BE = 2048
CE = 160
CK2 = 256
CB = 128
NBN = N // BN
NEG = -1e30

_mesh = None


def _get_mesh():
    global _mesh
    if _mesh is None:
        _mesh = plsc.VectorSubcoreMesh(
            core_axis_name="c", subcore_axis_name="s",
            num_cores=NC, num_subcores=NS)
    return _mesh


def _k1_body(batch_ref, x_ref, recipe_ref, Wx_ref, Wr_ref, b_ref, h_ref):
    bvals = batch_ref[0, 0, :]
    onehot = (bvals[:, None] == lax.broadcasted_iota(jnp.int32, (BN, G), 1))
    onehot = onehot.astype(jnp.float32)
    Rw = jnp.dot(recipe_ref[...], Wr_ref[...],
                 preferred_element_type=jnp.float32) + b_ref[...]
    acc = jnp.dot(x_ref[...], Wx_ref[...], preferred_element_type=jnp.float32)
    acc = acc + jnp.dot(onehot, Rw, preferred_element_type=jnp.float32)
    h_ref[...] = jnp.maximum(acc, 0.0)


def _k1(x, batch3d, recipe, Wx, Wr, b_in2d):
    return pl.pallas_call(
        _k1_body,
        grid=(NBN,),
        in_specs=[
            pl.BlockSpec((1, 1, BN), lambda i: (i, 0, 0)),
            pl.BlockSpec((BN, DN), lambda i: (i, 0)),
            pl.BlockSpec((G, DR), lambda i: (0, 0)),
            pl.BlockSpec((DN, HID), lambda i: (0, 0)),
            pl.BlockSpec((DR, HID), lambda i: (0, 0)),
            pl.BlockSpec((1, HID), lambda i: (0, 0)),
        ],
        out_specs=pl.BlockSpec((BN, HID), lambda i: (i, 0)),
        out_shape=jax.ShapeDtypeStruct((N, HID), jnp.float32),
    )(batch3d, x, recipe, Wx, Wr, b_in2d)


def _k1b_body(ea_ref, U_ref, c_ref, pay_ref, ae32_ref, me_ref):
    i = pl.program_id(0)
    ae = jnp.dot(ea_ref[...], U_ref[...],
                 preferred_element_type=jnp.float32) + c_ref[...]
    ones = jnp.ones((BE, 1), jnp.float32)
    pay_ref[...] = jnp.concatenate(
        [ae, ones, jnp.zeros((BE, 103), jnp.float32)], axis=1)
    ae32_ref[...] = jnp.concatenate(
        [ae, ones, jnp.zeros((BE, 7), jnp.float32)], axis=1)
    rows = []
    for l in range(L):
        m = jnp.max(ae[:, 8 * l:8 * l + 8], axis=0, keepdims=True)
        rows.append(jnp.concatenate(
            [m, jnp.full((1, 8), NEG, jnp.float32)], axis=1))
    blk = jnp.concatenate(rows, axis=0)

    @pl.when(i == 0)
    def _():
        me_ref[...] = jnp.full((L, 16), NEG, jnp.float32)

    me_ref[...] = jnp.maximum(me_ref[...], blk)


def _k1b(ea_p, U24, c24row):
    return pl.pallas_call(
        _k1b_body,
        grid=(E2 // BE,),
        in_specs=[
            pl.BlockSpec((BE, DE), lambda i: (i, 0)),
            pl.BlockSpec((DE, L * NH), lambda i: (0, 0)),
            pl.BlockSpec((1, L * NH), lambda i: (0, 0)),
        ],
        out_specs=[
            pl.BlockSpec((BE, 128), lambda i: (i, 0)),
            pl.BlockSpec((BE, 32), lambda i: (i, 0)),
            pl.BlockSpec((L, 16), lambda i: (0, 0)),
        ],
        out_shape=[
            jax.ShapeDtypeStruct((E2, 128), jnp.float32),
            jax.ShapeDtypeStruct((E2, 32), jnp.float32),
            jax.ShapeDtypeStruct((L, 16), jnp.float32),
        ],
    )(ea_p, U24, c24row)


def _k3_body(l, h_ref, W_ref, Asd_ref, me_ref, acc32a_ref, acc32b_ref,
             xh_ref, asd_ref, msd_ref, c16_ref):
    i = pl.program_id(0)
    j = pl.program_id(1)
    asd = jnp.dot(h_ref[...], Asd_ref[...], preferred_element_type=jnp.float32)
    asd_ref[...] = asd

    @pl.when(j < 4)
    def _():
        xh_ref[0] = jnp.dot(h_ref[...], W_ref[...],
                            preferred_element_type=jnp.float32)

    @pl.when(j == 4)
    def _():
        xh_ref[0] = jnp.concatenate(
            [asd, jnp.zeros((BN, 112), jnp.float32)], axis=1)

    a32 = acc32a_ref[0] + acc32b_ref[0]
    deg = jnp.maximum(a32[:, 24:25], 1.0)
    lae = a32[:, 8 * l:8 * l + 8] / deg
    mloop = jnp.max(lae, axis=0, keepdims=True)
    blk = jnp.max(asd, axis=0, keepdims=True)
    ml16 = jnp.concatenate([mloop, jnp.full((1, 8), NEG, jnp.float32)], axis=1)

    @pl.when((i == 0) & (j == 0))
    def _():
        msd_ref[...] = jnp.full((1, 32), NEG, jnp.float32)

    cur = msd_ref[...]
    cur = jnp.maximum(cur, jnp.concatenate([blk, ml16], axis=1))
    msd_ref[...] = cur
    ms = cur[:, 0:8]
    md = cur[:, 8:16]
    ml = cur[:, 16:24]
    me = me_ref[:, 0:8]
    c8 = ms + md + jnp.maximum(me, ml)
    c8 = jnp.where(c8 > 0, c8, 0.2 * c8)
    c16_ref[...] = jnp.concatenate(
        [c8, jnp.zeros((1, 8), jnp.float32)], axis=1)


def _k3(l, h, gW_l, Asd_l, me_l, acc32P):
    return pl.pallas_call(
        functools.partial(_k3_body, l),
        grid=(NBN, 5),
        in_specs=[
            pl.BlockSpec((BN, HID), lambda i, j: (i, 0)),
            pl.BlockSpec((HID, 128), lambda i, j: (0, lax.min(j, 3))),
            pl.BlockSpec((HID, 16), lambda i, j: (0, 0)),
            pl.BlockSpec((1, 16), lambda i, j: (0, 0)),
            pl.BlockSpec((1, BN, 128), lambda i, j: (0, i, 0)),
            pl.BlockSpec((1, BN, 128), lambda i, j: (1, i, 0)),
        ],
        out_specs=[
            pl.BlockSpec((1, BN, 128), lambda i, j: (j, i, 0)),
            pl.BlockSpec((BN, 16), lambda i, j: (i, 0)),
            pl.BlockSpec((1, 32), lambda i, j: (0, 0)),
            pl.BlockSpec((1, 16), lambda i, j: (0, 0)),
        ],
        out_shape=[
            jax.ShapeDtypeStruct((5, N, 128), jnp.float32),
            jax.ShapeDtypeStruct((N, 16), jnp.float32),
            jax.ShapeDtypeStruct((1, 32), jnp.float32),
            jax.ShapeDtypeStruct((1, 16), jnp.float32),
        ],
    )(h, gW_l, Asd_l, me_l, acc32P, acc32P)


def _k6_body(l, accPa_ref, accPb_ref, xh_ref, asd_ref,
             acc32a_ref, acc32b_ref, c16_ref, b_ref, h_ref):
    acc = [accPa_ref[0, c] + accPb_ref[0, c] for c in range(4)]
    xh = [xh_ref[c] for c in range(4)]
    dns = accPa_ref[0, 4] + accPb_ref[0, 4]

    a32 = acc32a_ref[0] + acc32b_ref[0]
    deg = jnp.maximum(a32[:, 24:25], 1.0)
    lae = a32[:, 8 * l:8 * l + 8] / deg
    asd = asd_ref[...]
    lself = asd[:, 0:8] + asd[:, 8:16] + lae
    lself = jnp.where(lself > 0, lself, 0.2 * lself)
    p_self = jnp.exp(lself - c16_ref[:, 0:8])
    denom = dns[:, 0:8] + p_self + 1e-16

    cols = []
    for h8 in range(NH):
        c = h8 // 2
        off = 64 * (h8 % 2)
        num = acc[c][:, off:off + 64] + xh[c][:, off:off + 64] * p_self[:, h8:h8 + 1]
        cols.append(num / denom[:, h8:h8 + 1])
    out = jnp.concatenate(cols, axis=1) + b_ref[...]
    h_ref[...] = jnp.maximum(out, 0.0)


def _k6(l, accP, xh5, asd, acc32P, c16, b_l2d):
    return pl.pallas_call(
        functools.partial(_k6_body, l),
        grid=(NBN,),
        in_specs=[
            pl.BlockSpec((1, 5, BN, 128), lambda i: (0, 0, i, 0)),
            pl.BlockSpec((1, 5, BN, 128), lambda i: (1, 0, i, 0)),
            pl.BlockSpec((4, BN, 128), lambda i: (0, i, 0)),
            pl.BlockSpec((BN, 16), lambda i: (i, 0)),
            pl.BlockSpec((1, BN, 128), lambda i: (0, i, 0)),
            pl.BlockSpec((1, BN, 128), lambda i: (1, i, 0)),
            pl.BlockSpec((1, 16), lambda i: (0, 0)),
            pl.BlockSpec((1, HID), lambda i: (0, 0)),
        ],
        out_specs=pl.BlockSpec((BN, HID), lambda i: (i, 0)),
        out_shape=jax.ShapeDtypeStruct((N, HID), jnp.float32),
    )(accP, accP, xh5, asd, acc32P, acc32P, c16, b_l2d)


def _k7_body(batch_ref, h_ref, W1_ref, b1_ref, W2_ref, b2_ref,
             g_ref, out_ref):
    i = pl.program_id(0)
    bvals = batch_ref[0, 0, :]
    onehot = (bvals[:, None] == lax.broadcasted_iota(jnp.int32, (BN, G), 1))
    onehot = onehot.astype(jnp.float32)
    gpart = lax.dot_general(onehot, h_ref[...], (((0,), (0,)), ((), ())),
                            preferred_element_type=jnp.float32)

    @pl.when(i == 0)
    def _():
        g_ref[...] = jnp.zeros((G, HID), jnp.float32)

    g = g_ref[...] + gpart
    g_ref[...] = g
    r = jnp.maximum(jnp.dot(g, W1_ref[...],
                            preferred_element_type=jnp.float32) + b1_ref[...],
                    0.0)
    out_ref[...] = jnp.dot(r, W2_ref[...],
                           preferred_element_type=jnp.float32) + b2_ref[...]


def _k7(h, batch3d, W_r1, b_r1_2d, W_r2p, b_r2p):
    return pl.pallas_call(
        _k7_body,
        grid=(NBN,),
        in_specs=[
            pl.BlockSpec((1, 1, BN), lambda i: (i, 0, 0)),
            pl.BlockSpec((BN, HID), lambda i: (i, 0)),
            pl.BlockSpec((HID, HID), lambda i: (0, 0)),
            pl.BlockSpec((1, HID), lambda i: (0, 0)),
            pl.BlockSpec((HID, 128), lambda i: (0, 0)),
            pl.BlockSpec((1, 128), lambda i: (0, 0)),
        ],
        out_specs=[
            pl.BlockSpec((G, HID), lambda i: (0, 0)),
            pl.BlockSpec((G, 128), lambda i: (0, 0)),
        ],
        out_shape=[
            jax.ShapeDtypeStruct((G, HID), jnp.float32),
            jax.ShapeDtypeStruct((G, 128), jnp.float32),
        ],
    )(batch3d, h, W_r1, b_r1_2d, W_r2p, b_r2p)


def _k2_body(pay_hbm, dst_hbm, zeros_hbm, out_hbm, pay, didx, sacc):
    cid = lax.axis_index("c")
    sid = lax.axis_index("s")
    wid = sid * NC + cid
    pltpu.sync_copy(zeros_hbm.at[pl.ds(sid * NPS, NPS)],
                    sacc.at[pl.ds(sid * NPS, NPS)])
    plsc.subcore_barrier()

    def chunk(k, _):
        base = wid * EPW + k * CK2
        pltpu.sync_copy(pay_hbm.at[pl.ds(base, CK2)], pay)
        pltpu.sync_copy(dst_hbm.at[pl.ds(base, CK2)], didx.at[0])
        pltpu.sync_copy(pay, sacc.at[didx.at[0]], add=True)
        return _

    lax.fori_loop(0, EPW // CK2, chunk, 0)
    plsc.subcore_barrier()
    pltpu.sync_copy(sacc.at[pl.ds(sid * NPS, NPS)],
                    out_hbm.at[cid, pl.ds(sid * NPS, NPS)])


def _k2(payload, dst_p, zeros128):
    f = pl.kernel(
        _k2_body,
        out_type=jax.ShapeDtypeStruct((NC, N2, 128), jnp.float32),
        mesh=_get_mesh(),
        compiler_params=pltpu.CompilerParams(needs_layout_passes=False),
        scratch_types=[
            pltpu.VMEM((CK2, 128), jnp.float32),
            pltpu.VMEM((1, CK2), jnp.int32),
            pltpu.VMEM_SHARED((N2, 128), jnp.float32),
        ],
    )
    return f(payload, dst_p, zeros128)


def _k4_body(l, xh_hbm, pay_hbm, src_hbm, dst_hbm, c16_hbm,
             p_hbm, sb, abuf, pay, pnar, sidx, didx, gs, gd,
             cvec, sem):
    cid = lax.axis_index("c")
    sid = lax.axis_index("s")
    wid = sid * NC + cid
    pltpu.sync_copy(c16_hbm.at[0], cvec)

    iota = lax.iota(jnp.int32, 16)
    chs = [plsc.load_gather(cvec, [jnp.full((16,), h, jnp.int32)])
           for h in range(NH)]

    def chunk(k, _):
        base = wid * EPW + k * CE
        pltpu.sync_copy(src_hbm.at[pl.ds(base, CE)], sidx)
        pltpu.sync_copy(dst_hbm.at[pl.ds(base, CE)], didx)
        pltpu.sync_copy(pay_hbm.at[pl.ds(base, CE)], pay)

        def shift(g, _2):
            gs[pl.ds(g * 16, 16)] = sidx[pl.ds(g * 16, 16)] + 4 * N
            gd[pl.ds(g * 16, 16)] = didx[pl.ds(g * 16, 16)] + 4 * N
            return _2

        lax.fori_loop(0, CE // 16, shift, 0)

        pltpu.async_copy(xh_hbm.at[gs], sb, sem).wait()
        for h in range(NH):
            hh = jnp.full((16,), h, jnp.int32)

            def ext_s(g, _2):
                row = g * 16 + iota
                plsc.store_scatter(abuf, [row, hh],
                                   plsc.load_gather(sb, [row, hh]))
                return _2

            lax.fori_loop(0, CE // 16, ext_s, 0)

        pltpu.async_copy(xh_hbm.at[gd], sb, sem).wait()
        for h in range(NH):
            hd = jnp.full((16,), 8 + h, jnp.int32)

            def ext_d(g, _2):
                row = g * 16 + iota
                plsc.store_scatter(abuf, [row, hd],
                                   plsc.load_gather(sb, [row, hd]))
                return _2

            lax.fori_loop(0, CE // 16, ext_d, 0)

        for h in range(NH):
            hsrc = jnp.full((16,), h, jnp.int32)
            hdst = jnp.full((16,), 8 + h, jnp.int32)
            hae = jnp.full((16,), 8 * l + h, jnp.int32)
            ch = chs[h]

            def grp(g, _2):
                row = g * 16 + iota
                a_s = plsc.load_gather(abuf, [row, hsrc])
                a_d = plsc.load_gather(abuf, [row, hdst])
                ae = plsc.load_gather(pay, [row, hae])
                t = a_s + a_d + ae
                t = jnp.where(t > 0, t, 0.2 * t)
                pv = jnp.exp(t - ch)
                plsc.store_scatter(pnar, [row, hsrc], pv)
                return _2

            lax.fori_loop(0, CE // 16, grp, 0)

        pltpu.sync_copy(pnar, p_hbm.at[pl.ds(base, CE)])
        return _

    lax.fori_loop(0, EPW // CE, chunk, 0)


def _k4(l, xh5flat, ae32, src_p, dst_p, c16):
    f = pl.kernel(
        functools.partial(_k4_body, l),
        out_type=jax.ShapeDtypeStruct((E2, 16), jnp.float32),
        mesh=_get_mesh(),
        compiler_params=pltpu.CompilerParams(needs_layout_passes=False),
        scratch_types=[
            pltpu.VMEM((CE, 128), jnp.float32),
            pltpu.VMEM((CE, 16), jnp.float32),
            pltpu.VMEM((CE, 32), jnp.float32),
            pltpu.VMEM((CE, 16), jnp.float32),
            pltpu.VMEM((CE,), jnp.int32),
            pltpu.VMEM((CE,), jnp.int32),
            pltpu.VMEM((CE,), jnp.int32),
            pltpu.VMEM((CE,), jnp.int32),
            pltpu.VMEM((16,), jnp.float32),
            pltpu.SemaphoreType.DMA,
        ],
    )
    return f(xh5flat, ae32, src_p, dst_p, c16)


def _k5_body(xh_hbm, p_hbm, src_hbm, dst_hbm, zeros_hbm, out_hbm,
             rows, pch, sidx, didx, gidx, sacc, sem):
    cid = lax.axis_index("c")
    sid = lax.axis_index("s")
    wid = sid * NC + cid

    for cc in range(5):
        pltpu.sync_copy(zeros_hbm.at[pl.ds(sid * NPS, NPS)],
                        sacc.at[pl.ds(sid * NPS, NPS)])
        plsc.subcore_barrier()

        if cc < 4:
            def chunk(k, _):
                base = wid * EPW + k * CB
                pltpu.sync_copy(src_hbm.at[pl.ds(base, CB)], sidx)
                pltpu.sync_copy(dst_hbm.at[pl.ds(base, CB)], didx)
                pltpu.sync_copy(p_hbm.at[pl.ds(base, CB)], pch)

                def shift(g, _2):
                    gidx[pl.ds(g * 16, 16)] = sidx[pl.ds(g * 16, 16)] + cc * N
                    return _2

                lax.fori_loop(0, CB // 16, shift, 0)
                pltpu.async_copy(xh_hbm.at[gidx], rows, sem).wait()

                def edge(e, _2):
                    p0 = plsc.load_gather(
                        pch, [jnp.full((16,), e, jnp.int32),
                              jnp.full((16,), 2 * cc, jnp.int32)])
                    p1 = plsc.load_gather(
                        pch, [jnp.full((16,), e, jnp.int32),
                              jnp.full((16,), 2 * cc + 1, jnp.int32)])
                    for j in range(8):
                        pj = p0 if j < 4 else p1
                        rows[e, pl.ds(j * 16, 16)] = (
                            rows[e, pl.ds(j * 16, 16)] * pj)
                    return _2

                lax.fori_loop(0, CB, edge, 0)
                pltpu.sync_copy(rows, sacc.at[didx], add=True)
                return _

            lax.fori_loop(0, EPW // CB, chunk, 0)
        else:
            def chunk4(k, _):
                base = wid * EPW + k * CB
                pltpu.sync_copy(dst_hbm.at[pl.ds(base, CB)], didx)
                pltpu.sync_copy(p_hbm.at[pl.ds(base, CB)], pch)

                def edge4(e, _2):
                    rows[e, pl.ds(0, 16)] = pch[e, :]
                    return _2

                lax.fori_loop(0, CB, edge4, 0)
                pltpu.sync_copy(rows, sacc.at[didx], add=True)
                return _

            lax.fori_loop(0, EPW // CB, chunk4, 0)

        plsc.subcore_barrier()
        pltpu.sync_copy(sacc.at[pl.ds(sid * NPS, NPS)],
                        out_hbm.at[cid, cc, pl.ds(sid * NPS, NPS)])
        plsc.subcore_barrier()


def _k5(xh4flat, p, src_p, dst_p, zeros128):
    f = pl.kernel(
        _k5_body,
        out_type=jax.ShapeDtypeStruct((NC, 5, N2, 128), jnp.float32),
        mesh=_get_mesh(),
        compiler_params=pltpu.CompilerParams(needs_layout_passes=False),
        scratch_types=[
            pltpu.VMEM((CB, 128), jnp.float32),
            pltpu.VMEM((CB, 16), jnp.float32),
            pltpu.VMEM((CB,), jnp.int32),
            pltpu.VMEM((CB,), jnp.int32),
            pltpu.VMEM((CB,), jnp.int32),
            pltpu.VMEM_SHARED((N2, 128), jnp.float32),
            pltpu.SemaphoreType.DMA,
        ],
    )
    return f(xh4flat, p, src_p, dst_p, zeros128)


def kernel(x, edge_index, edge_attr, recipe, batch, W_in, b_in, W_ee, b_ee,
           gat_W, att_src, att_dst, gat_We, att_edge, gat_b, W_r1, b_r1,
           W_r2, b_r2):
    f32 = jnp.float32
    gw = gat_W.reshape(L, HID, NH, HC)
    A_s = jnp.einsum('lkhc,lhc->lkh', gw, att_src)
    A_d = jnp.einsum('lkhc,lhc->lkh', gw, att_dst)
    A_sd = jnp.concatenate([A_s, A_d], axis=-1)
    V = jnp.einsum('lkhc,lhc->lkh', gat_We.reshape(L, HID, NH, HC), att_edge)
    U = jnp.einsum('dk,lkh->ldh', W_ee, V)
    U24 = jnp.moveaxis(U, 0, 1).reshape(DE, L * NH)
    c24row = jnp.einsum('k,lkh->lh', b_ee, V).reshape(1, L * NH)

    Wx = W_in[:DN]
    Wr = W_in[DN:]
    b_in2d = b_in.reshape(1, HID)
    batch3d = batch.astype(jnp.int32).reshape(NBN, 1, BN)
    W_r2p = jnp.pad(W_r2, ((0, 0), (0, 127)))
    b_r2p = jnp.pad(b_r2.reshape(1, 1), ((0, 0), (0, 127)))

    src_p = jnp.concatenate(
        [edge_index[0].astype(jnp.int32), jnp.zeros((E2 - E,), jnp.int32)])
    dst_p = jnp.concatenate(
        [edge_index[1].astype(jnp.int32), jnp.full((E2 - E,), N, jnp.int32)])
    ea_p = jnp.concatenate(
        [edge_attr, jnp.zeros((E2 - E, DE), f32)], axis=0)

    zeros128 = jnp.zeros((N2, 128), f32)

    h = _k1(x, batch3d, recipe, Wx, Wr, b_in2d)
    payload, ae32, m_e = _k1b(ea_p, U24, c24row)
    acc32P = _k2(payload, dst_p, zeros128)

    for l in range(L):
        xh5, asd, _msd, c16 = _k3(l, h, gat_W[l], A_sd[l],
                                  m_e[l:l + 1], acc32P)
        xh5flat = xh5.reshape(5 * N, 128)
        p = _k4(l, xh5flat, ae32, src_p, dst_p, c16)
        accP = _k5(xh5flat, p, src_p, dst_p, zeros128)
        h = _k6(l, accP, xh5, asd, acc32P, c16,
                gat_b[l].reshape(1, HID))

    _g, out = _k7(h, batch3d, W_r1, b_r1.reshape(1, HID), W_r2p, b_r2p)
    return out[:, :1]
